# baseline (speedup 1.0000x reference)
import functools

import jax
import jax.numpy as jnp
from jax import lax
from jax.experimental import pallas as pl
from jax.experimental.pallas import tpu as pltpu
from jax.experimental.pallas import tpu_sc as plsc

B = 4096
L = 50
LP = 56
C = 100
D = 64
V = 1000001

NC = 2
NS = 16
NW = NC * NS
BPW = B // NW

SEQ_PAIR = 2
SEQ_IDX = SEQ_PAIR * LP
SEQ_ITERS = BPW // SEQ_PAIR

CAND_PW = (B * C) // NW
CCH = 128
CAND_ITERS = CAND_PW // CCH

_sc_mesh = plsc.VectorSubcoreMesh(core_axis_name="c", subcore_axis_name="s")


@functools.partial(
    pl.kernel,
    mesh=_sc_mesh,
    out_type=[
        jax.ShapeDtypeStruct((B, D), jnp.float32),
        jax.ShapeDtypeStruct((B * C, D), jnp.float32),
    ],
    scratch_types=[
        pltpu.VMEM((SEQ_IDX,), jnp.int32),
        pltpu.VMEM((SEQ_IDX, D), jnp.float32),
        pltpu.VMEM((BPW, D), jnp.float32),
        pltpu.VMEM((CCH,), jnp.int32),
        pltpu.VMEM((CCH, D), jnp.float32),
        pltpu.SemaphoreType.DMA,
        pltpu.SemaphoreType.DMA,
    ],
    compiler_params=pltpu.CompilerParams(use_tc_tiling_on_sc=False),
)
def _sc_gather(seq_hbm, cand_hbm, table_hbm, summed_hbm, cand_emb_hbm,
               idx_s, rows_s, acc_v, idx_c, rows_c, sem_s, sem_c):
    wid = lax.axis_index("s") * NC + lax.axis_index("c")

    def seq_step(g, carry):
        base_row = wid * BPW + g * SEQ_PAIR
        pltpu.sync_copy(seq_hbm.at[pl.ds(base_row * LP, SEQ_IDX)], idx_s)
        pltpu.async_copy(table_hbm.at[idx_s], rows_s, sem_s).wait()
        for rr in range(SEQ_PAIR):
            accs = [rows_s[rr * LP, pl.ds(dd * 16, 16)] for dd in range(4)]
            for l in range(1, LP):
                for dd in range(4):
                    accs[dd] = accs[dd] + rows_s[rr * LP + l, pl.ds(dd * 16, 16)]
            for dd in range(4):
                acc_v[g * SEQ_PAIR + rr, pl.ds(dd * 16, 16)] = accs[dd]
        return carry

    lax.fori_loop(0, SEQ_ITERS, seq_step, 0)
    pltpu.sync_copy(acc_v, summed_hbm.at[pl.ds(wid * BPW, BPW)])

    def cand_step(ci, carry):
        base = wid * CAND_PW + ci * CCH
        pltpu.sync_copy(cand_hbm.at[pl.ds(base, CCH)], idx_c)
        pltpu.async_copy(table_hbm.at[idx_c], rows_c, sem_c).wait()
        pltpu.sync_copy(rows_c, cand_emb_hbm.at[pl.ds(base, CCH)])
        return carry

    lax.fori_loop(0, CAND_ITERS, cand_step, 0)


TB = 64


def _tc_body(seq_ref, summed_ref, cand_ref, ug_ref, ub_ref, ig_ref, ib_ref,
             out_ref):
    seq = seq_ref[...]
    cnt = jnp.sum((seq != 0).astype(jnp.float32), axis=1, keepdims=True)
    cnt = jnp.maximum(cnt, 1.0)
    pooled = summed_ref[...] / cnt

    m = jnp.mean(pooled, axis=1, keepdims=True)
    xc = pooled - m
    v = jnp.mean(xc * xc, axis=1, keepdims=True)
    y = xc * lax.rsqrt(v + 1e-5) * ug_ref[...] + ub_ref[...]
    n = jnp.sqrt(jnp.sum(y * y, axis=1, keepdims=True))
    user = y / jnp.maximum(n, 1e-12)

    cand = cand_ref[...]
    cm = jnp.mean(cand, axis=2, keepdims=True)
    cc = cand - cm
    cv = jnp.mean(cc * cc, axis=2, keepdims=True)
    cy = cc * lax.rsqrt(cv + 1e-5) * ig_ref[...][None] + ib_ref[...][None]
    cn = jnp.sqrt(jnp.sum(cy * cy, axis=2, keepdims=True))
    cvec = cy / jnp.maximum(cn, 1e-12)

    out_ref[...] = jnp.sum(cvec * user[:, None, :], axis=2)


_tc_call = pl.pallas_call(
    _tc_body,
    grid=(B // TB,),
    in_specs=[
        pl.BlockSpec((TB, L), lambda i: (i, 0)),
        pl.BlockSpec((TB, D), lambda i: (i, 0)),
        pl.BlockSpec((TB, C, D), lambda i: (i, 0, 0)),
        pl.BlockSpec((1, D), lambda i: (0, 0)),
        pl.BlockSpec((1, D), lambda i: (0, 0)),
        pl.BlockSpec((1, D), lambda i: (0, 0)),
        pl.BlockSpec((1, D), lambda i: (0, 0)),
    ],
    out_specs=pl.BlockSpec((TB, C), lambda i: (i, 0)),
    out_shape=jax.ShapeDtypeStruct((B, C), jnp.float32),
)


def kernel(seq, candidate_item_ids, item_embedding, user_norm_g, user_norm_b,
           item_norm_g, item_norm_b):
    seq = seq.astype(jnp.int32)
    cand = candidate_item_ids.astype(jnp.int32)
    seq_p = jnp.pad(seq, ((0, 0), (0, LP - L)))
    summed, cand_emb = _sc_gather(
        seq_p.reshape(-1), cand.reshape(-1), item_embedding)
    scores = _tc_call(
        seq, summed, cand_emb.reshape(B, C, D),
        user_norm_g.reshape(1, D), user_norm_b.reshape(1, D),
        item_norm_g.reshape(1, D), item_norm_b.reshape(1, D))
    return scores
